# baseline (speedup 1.0000x reference)
import functools
import numpy as np
import jax
import jax.numpy as jnp
from jax import lax
from jax.experimental import pallas as pl
from jax.experimental.pallas import tpu as pltpu
from jax.experimental.pallas import tpu_sc as plsc

NUM_CAMERA = 100000
BATCH = 16384
C_ROT = np.float32(5.0 / 180.0 * np.pi)
C_TR = np.float32(0.1)

NC, NS, L = 2, 16, 16
NW = NC * NS
BPW = BATCH // NW
NH = 4
HPW = BPW // NH
HCH = HPW // L


def _tanh(x):
    e = jnp.exp(-2.0 * jnp.abs(x))
    return jnp.sign(x) * (1.0 - e) / (1.0 + e)


def _body(rot_hbm, tr_hbm, idx_hbm, out_hbm,
          i0, rx, ry, rz, tx, ty, tz, stage,
          isem, s0, s1, s2, s3):
    sems = (s0, s1, s2, s3)
    wid = lax.axis_index("s") * NC + lax.axis_index("c")
    base = wid * BPW

    pltpu.sync_copy(idx_hbm.at[pl.ds(base, BPW)], i0)
    N = NUM_CAMERA
    copies = [
        pltpu.async_copy(rot_hbm.at[pl.ds(0, N)].at[i0], rx, sems[0]),
        pltpu.async_copy(rot_hbm.at[pl.ds(N, N)].at[i0], ry, sems[1]),
        pltpu.async_copy(rot_hbm.at[pl.ds(2 * N, N)].at[i0], rz, sems[2]),
        pltpu.async_copy(tr_hbm.at[pl.ds(0, N)].at[i0], tx, sems[3]),
        pltpu.async_copy(tr_hbm.at[pl.ds(N, N)].at[i0], ty, sems[0]),
        pltpu.async_copy(tr_hbm.at[pl.ds(2 * N, N)].at[i0], tz, sems[1]),
    ]

    zero = jnp.zeros((L,), jnp.float32)
    one = jnp.ones((L,), jnp.float32)

    def chunk(k, _):
        s = pl.ds(k * L, L)
        ax = C_ROT * _tanh(rx[s])
        ay = C_ROT * _tanh(ry[s])
        az = C_ROT * _tanh(rz[s])
        vtx = C_TR * _tanh(tx[s])
        vty = C_TR * _tanh(ty[s])
        vtz = C_TR * _tanh(tz[s])

        x2, y2, z2 = ax * ax, ay * ay, az * az
        t2 = x2 + y2 + z2
        a = 1.0 + t2 * (-1.0 / 6.0 + t2 * (1.0 / 120.0 + t2 * (-1.0 / 5040.0)))
        b = 0.5 + t2 * (-1.0 / 24.0 + t2 * (1.0 / 720.0 + t2 * (-1.0 / 40320.0)))

        bxy, bxz, byz = b * ax * ay, b * ax * az, b * ay * az
        aX, aY, aZ = a * ax, a * ay, a * az
        vals = (
            1.0 - b * (y2 + z2), bxy - aZ, bxz + aY, vtx,
            bxy + aZ, 1.0 - b * (x2 + z2), byz - aX, vty,
            bxz - aY, byz + aX, 1.0 - b * (x2 + y2), vtz,
            zero, zero, zero, one,
        )
        for p, v in enumerate(vals):
            stage[p, s] = v
        return ()

    for cp in copies:
        cp.wait()
    lax.fori_loop(0, BPW // L, chunk, (), unroll=4)
    pltpu.sync_copy(stage, out_hbm.at[:, pl.ds(base, BPW)])


@jax.jit
def _run(rotations, translations, camera_idx):
    mesh = plsc.VectorSubcoreMesh(
        core_axis_name="c", subcore_axis_name="s", num_cores=NC, num_subcores=NS
    )
    f = pl.kernel(
        _body,
        out_type=jax.ShapeDtypeStruct((16, BATCH), jnp.float32),
        mesh=mesh,
        compiler_params=pltpu.CompilerParams(needs_layout_passes=False),
        scratch_types=[
            pltpu.VMEM((BPW,), jnp.int32),
            pltpu.VMEM((BPW,), jnp.float32),
            pltpu.VMEM((BPW,), jnp.float32),
            pltpu.VMEM((BPW,), jnp.float32),
            pltpu.VMEM((BPW,), jnp.float32),
            pltpu.VMEM((BPW,), jnp.float32),
            pltpu.VMEM((BPW,), jnp.float32),
            pltpu.VMEM((16, BPW), jnp.float32),
            pltpu.SemaphoreType.DMA,
            pltpu.SemaphoreType.DMA,
            pltpu.SemaphoreType.DMA,
            pltpu.SemaphoreType.DMA,
            pltpu.SemaphoreType.DMA,
        ],
    )
    rot_t = jnp.transpose(rotations).reshape(-1)
    tr_t = jnp.transpose(translations).reshape(-1)
    out = f(rot_t, tr_t, camera_idx)
    return out.reshape(4, 4, BATCH).transpose(2, 0, 1)


def kernel(rotations, translations, camera_idx):
    return _run(rotations, translations, camera_idx.astype(jnp.int32))


# speedup vs baseline: 1.0051x; 1.0051x over previous
import functools
import numpy as np
import jax
import jax.numpy as jnp
from jax import lax
from jax.experimental import pallas as pl
from jax.experimental.pallas import tpu as pltpu
from jax.experimental.pallas import tpu_sc as plsc

NUM_CAMERA = 100000
BATCH = 16384
C_ROT = np.float32(5.0 / 180.0 * np.pi)
C_TR = np.float32(0.1)

NC, NS, L = 2, 16, 16
NW = NC * NS
BPW = BATCH // NW
NH = 2
HPW = BPW // NH
HCH = HPW // L


def _tanh(x):
    e = jnp.exp(-2.0 * jnp.abs(x))
    return jnp.sign(x) * (1.0 - e) / (1.0 + e)


def _body(rot_hbm, tr_hbm, idx_hbm, out_hbm,
          i0, rxa, rya, rza, txa, tya, tza, rxb, ryb, rzb, txb, tyb, tzb,
          stage, isem, s0, s1, s2, s3):
    wid = lax.axis_index("s") * NC + lax.axis_index("c")
    base = wid * BPW

    pltpu.sync_copy(idx_hbm.at[pl.ds(base, BPW)], i0)
    N = NUM_CAMERA
    views = (rot_hbm.at[pl.ds(0, N)], rot_hbm.at[pl.ds(N, N)],
             rot_hbm.at[pl.ds(2 * N, N)], tr_hbm.at[pl.ds(0, N)],
             tr_hbm.at[pl.ds(N, N)], tr_hbm.at[pl.ds(2 * N, N)])
    bufs = ((rxa, rya, rza, txa, tya, tza), (rxb, ryb, rzb, txb, tyb, tzb))
    sems = (s0, s1)
    copies = [
        [pltpu.async_copy(views[c].at[i0.at[pl.ds(h * HPW, HPW)]],
                          bufs[h][c], sems[h])
         for c in range(6)]
        for h in range(NH)
    ]

    zero = jnp.zeros((L,), jnp.float32)
    one = jnp.ones((L,), jnp.float32)

    def make_chunk(h):
        rx, ry, rz, tx, ty, tz = bufs[h]

        def chunk(k, _):
            c = pl.ds(k * L, L)
            s = pl.ds(h * HPW + k * L, L)
            ax = C_ROT * _tanh(rx[c])
            ay = C_ROT * _tanh(ry[c])
            az = C_ROT * _tanh(rz[c])
            vtx = C_TR * _tanh(tx[c])
            vty = C_TR * _tanh(ty[c])
            vtz = C_TR * _tanh(tz[c])

            x2, y2, z2 = ax * ax, ay * ay, az * az
            t2 = x2 + y2 + z2
            a = 1.0 + t2 * (-1.0 / 6.0 + t2 * (1.0 / 120.0 + t2 * (-1.0 / 5040.0)))
            b = 0.5 + t2 * (-1.0 / 24.0 + t2 * (1.0 / 720.0 + t2 * (-1.0 / 40320.0)))

            bxy, bxz, byz = b * ax * ay, b * ax * az, b * ay * az
            aX, aY, aZ = a * ax, a * ay, a * az
            vals = (
                1.0 - b * (y2 + z2), bxy - aZ, bxz + aY, vtx,
                bxy + aZ, 1.0 - b * (x2 + z2), byz - aX, vty,
                bxz - aY, byz + aX, 1.0 - b * (x2 + y2), vtz,
                zero, zero, zero, one,
            )
            for p, v in enumerate(vals):
                stage[p, s] = v
            return ()
        return chunk

    out_cp = []
    for h in range(NH):
        for cp in copies[h]:
            cp.wait()
        lax.fori_loop(0, HPW // L, make_chunk(h), (), unroll=2)
        out_cp.append(pltpu.async_copy(
            stage.at[:, pl.ds(h * HPW, HPW)],
            out_hbm.at[:, pl.ds(base + h * HPW, HPW)], s3))
    for cp in out_cp:
        cp.wait()


@jax.jit
def _run(rotations, translations, camera_idx):
    mesh = plsc.VectorSubcoreMesh(
        core_axis_name="c", subcore_axis_name="s", num_cores=NC, num_subcores=NS
    )
    f = pl.kernel(
        _body,
        out_type=jax.ShapeDtypeStruct((16, BATCH), jnp.float32),
        mesh=mesh,
        compiler_params=pltpu.CompilerParams(needs_layout_passes=False),
        scratch_types=[
            pltpu.VMEM((BPW,), jnp.int32),
        ] + [pltpu.VMEM((HPW,), jnp.float32)] * 12 + [
            pltpu.VMEM((16, BPW), jnp.float32),
            pltpu.SemaphoreType.DMA,
            pltpu.SemaphoreType.DMA,
            pltpu.SemaphoreType.DMA,
            pltpu.SemaphoreType.DMA,
            pltpu.SemaphoreType.DMA,
        ],
    )
    rot_t = jnp.transpose(rotations).reshape(-1)
    tr_t = jnp.transpose(translations).reshape(-1)
    out = f(rot_t, tr_t, camera_idx)
    return out.reshape(4, 4, BATCH).transpose(2, 0, 1)


def kernel(rotations, translations, camera_idx):
    return _run(rotations, translations, camera_idx.astype(jnp.int32))


# speedup vs baseline: 1.0143x; 1.0091x over previous
import functools
import numpy as np
import jax
import jax.numpy as jnp
from jax import lax
from jax.experimental import pallas as pl
from jax.experimental.pallas import tpu as pltpu
from jax.experimental.pallas import tpu_sc as plsc

NUM_CAMERA = 100000
BATCH = 16384
C_ROT = np.float32(5.0 / 180.0 * np.pi)
C_TR = np.float32(0.1)

NC, NS, L = 2, 16, 16
NW = NC * NS
BPW = BATCH // NW


def _tanh(x):
    e = jnp.exp(-2.0 * jnp.abs(x))
    return jnp.sign(x) * (1.0 - e) / (1.0 + e)


def _body(rot_hbm, tr_hbm, idx_hbm, out_hbm,
          i0, rx, ry, rz, tx, ty, tz, stage, s0, s1, s2, s3):
    sems = (s0, s1, s2, s3)
    wid = lax.axis_index("s") * NC + lax.axis_index("c")
    base = wid * BPW

    pltpu.sync_copy(idx_hbm.at[pl.ds(base, BPW)], i0)
    N = NUM_CAMERA
    copies = [
        pltpu.async_copy(rot_hbm.at[pl.ds(0, N)].at[i0], rx, sems[0]),
        pltpu.async_copy(rot_hbm.at[pl.ds(N, N)].at[i0], ry, sems[1]),
        pltpu.async_copy(rot_hbm.at[pl.ds(2 * N, N)].at[i0], rz, sems[2]),
        pltpu.async_copy(tr_hbm.at[pl.ds(0, N)].at[i0], tx, sems[3]),
        pltpu.async_copy(tr_hbm.at[pl.ds(N, N)].at[i0], ty, sems[0]),
        pltpu.async_copy(tr_hbm.at[pl.ds(2 * N, N)].at[i0], tz, sems[1]),
    ]

    zero = jnp.zeros((L,), jnp.float32)
    one = jnp.ones((L,), jnp.float32)

    def chunk(k, _):
        s = pl.ds(k * L, L)
        ax = C_ROT * _tanh(rx[s])
        ay = C_ROT * _tanh(ry[s])
        az = C_ROT * _tanh(rz[s])
        vtx = C_TR * _tanh(tx[s])
        vty = C_TR * _tanh(ty[s])
        vtz = C_TR * _tanh(tz[s])

        x2, y2, z2 = ax * ax, ay * ay, az * az
        t2 = x2 + y2 + z2
        a = 1.0 + t2 * (-1.0 / 6.0 + t2 * (1.0 / 120.0 + t2 * (-1.0 / 5040.0)))
        b = 0.5 + t2 * (-1.0 / 24.0 + t2 * (1.0 / 720.0 + t2 * (-1.0 / 40320.0)))

        bxy, bxz, byz = b * ax * ay, b * ax * az, b * ay * az
        aX, aY, aZ = a * ax, a * ay, a * az
        vals = (
            1.0 - b * (y2 + z2), bxy - aZ, bxz + aY, vtx,
            bxy + aZ, 1.0 - b * (x2 + z2), byz - aX, vty,
            bxz - aY, byz + aX, 1.0 - b * (x2 + y2), vtz,
            zero, zero, zero, one,
        )
        for p, v in enumerate(vals):
            stage[p, s] = v
        return ()

    for cp in copies:
        cp.wait()
    lax.fori_loop(0, BPW // L, chunk, (), unroll=2)
    pltpu.sync_copy(stage, out_hbm.at[:, pl.ds(base, BPW)])


@jax.jit
def _run(rotations, translations, camera_idx):
    mesh = plsc.VectorSubcoreMesh(
        core_axis_name="c", subcore_axis_name="s", num_cores=NC, num_subcores=NS
    )
    f = pl.kernel(
        _body,
        out_type=jax.ShapeDtypeStruct((16, BATCH), jnp.float32),
        mesh=mesh,
        compiler_params=pltpu.CompilerParams(needs_layout_passes=False),
        scratch_types=[
            pltpu.VMEM((BPW,), jnp.int32),
            pltpu.VMEM((BPW,), jnp.float32),
            pltpu.VMEM((BPW,), jnp.float32),
            pltpu.VMEM((BPW,), jnp.float32),
            pltpu.VMEM((BPW,), jnp.float32),
            pltpu.VMEM((BPW,), jnp.float32),
            pltpu.VMEM((BPW,), jnp.float32),
            pltpu.VMEM((16, BPW), jnp.float32),
            pltpu.SemaphoreType.DMA,
            pltpu.SemaphoreType.DMA,
            pltpu.SemaphoreType.DMA,
            pltpu.SemaphoreType.DMA,
        ],
    )
    rot_t = jnp.transpose(rotations).reshape(-1)
    tr_t = jnp.transpose(translations).reshape(-1)
    out = f(rot_t, tr_t, camera_idx)
    return out.reshape(4, 4, BATCH).transpose(2, 0, 1)


def kernel(rotations, translations, camera_idx):
    return _run(rotations, translations, camera_idx.astype(jnp.int32))
